# baseline (speedup 1.0000x reference)
import jax
import jax.numpy as jnp
from jax.experimental import pallas as pl
from jax.experimental.pallas import tpu as pltpu

N = 32768
B = 16
D = 128
R = 4096
T = N // R


def _body(x_hbm, o_ref, w1_ref, b1_ref, gamma_ref, beta_ref, w2_ref, b2_ref,
          out_ref, gram_ref, segsum_ref, ws_ref, f_ref, xbuf_ref, ohbuf_ref,
          sems):
    i = pl.program_id(0)
    phase_a = i < T
    t = jnp.where(phase_a, i, i - T)

    o_col = o_ref[...]
    op_col = jnp.concatenate(
        [jnp.zeros((1, 1), jnp.int32), o_col[:-1, :]], axis=0)

    @pl.when(phase_a)
    def _accum():
        @pl.when(i == 0)
        def _init():
            gram_ref[...] = jnp.zeros_like(gram_ref)
            segsum_ref[...] = jnp.zeros_like(segsum_ref)
            for k in range(T):
                pltpu.make_async_copy(
                    x_hbm.at[pl.ds(k * R, R), :],
                    xbuf_ref.at[pl.ds(k * R, R), :],
                    sems.at[k]).start()

        for k in range(T):
            @pl.when(i == k)
            def _wait():
                pltpu.make_async_copy(
                    x_hbm.at[pl.ds(k * R, R), :],
                    xbuf_ref.at[pl.ds(k * R, R), :],
                    sems.at[k]).wait()

        base = i * R
        r = jax.lax.broadcasted_iota(jnp.int32, (B, R), 1)
        oh_t = ((r >= op_col - base) & (r < o_col - base)).astype(jnp.float32)
        ohbuf_ref[:, pl.ds(i * R, R)] = oh_t
        x = xbuf_ref[pl.ds(i * R, R), :]
        gram_ref[...] += jax.lax.dot_general(
            x, x, (((0,), (0,)), ((), ())), preferred_element_type=jnp.float32)
        segsum_ref[...] += jnp.dot(oh_t, x, preferred_element_type=jnp.float32)

    @pl.when(i == T)
    def _mid():
        cnt = (o_col - op_col).astype(jnp.float32)
        segsum = segsum_ref[...]
        w1a = w1_ref[:D, :]
        seg_mean = segsum / jnp.maximum(cnt, 1.0)
        g = jax.nn.relu(jnp.dot(seg_mean, w2_ref[...],
                                preferred_element_type=jnp.float32)
                        + b2_ref[...])
        e = jnp.dot(g, w1_ref[D:, :],
                    preferred_element_type=jnp.float32) + b1_ref[...]
        segsum_a = jnp.dot(segsum, w1a, preferred_element_type=jnp.float32)
        sum_a2 = jnp.sum(w1a * jnp.dot(gram_ref[...], w1a,
                                       preferred_element_type=jnp.float32),
                         axis=0, keepdims=True)
        sum_h = jnp.sum(segsum_a + cnt * e, axis=0, keepdims=True)
        sum_h2 = sum_a2 + jnp.sum(2.0 * e * segsum_a + cnt * e * e,
                                  axis=0, keepdims=True)
        mu = sum_h / N
        var = sum_h2 / N - mu * mu
        scale = gamma_ref[...] * jax.lax.rsqrt(var + 1e-5)
        shift = beta_ref[...] - mu * scale
        ws_ref[...] = w1a * scale
        f_ref[...] = e * scale + shift

    @pl.when(jnp.logical_not(phase_a))
    def _apply():
        xb = xbuf_ref[pl.ds(t * R, R), :]
        a = jnp.dot(xb, ws_ref[...], preferred_element_type=jnp.float32)
        seg_f = jax.lax.dot_general(
            ohbuf_ref[:, pl.ds(t * R, R)], f_ref[...], (((0,), (0,)), ((), ())),
            preferred_element_type=jnp.float32)
        out_ref[...] = jax.nn.relu(a + seg_f)


def kernel(p, x, o, W1, b1, gamma, beta, W2, b2):
    del p
    full = lambda shape: pl.BlockSpec(shape, lambda *_: (0,) * len(shape))
    out_spec = pl.BlockSpec((R, D), lambda i: (jnp.where(i < T, 0, i - T), 0))

    return pl.pallas_call(
        _body,
        grid=(2 * T,),
        in_specs=[
            pl.BlockSpec(memory_space=pl.ANY),
            full((B, 1)), full((2 * D, D)), full((1, D)), full((1, D)),
            full((1, D)), full((D, D)), full((1, D)),
        ],
        out_specs=out_spec,
        out_shape=jax.ShapeDtypeStruct((N, D), jnp.float32),
        scratch_shapes=[
            pltpu.VMEM((D, D), jnp.float32),
            pltpu.VMEM((B, D), jnp.float32),
            pltpu.VMEM((D, D), jnp.float32),
            pltpu.VMEM((B, D), jnp.float32),
            pltpu.VMEM((N, D), jnp.float32),
            pltpu.VMEM((B, N), jnp.float32),
            pltpu.SemaphoreType.DMA((T,)),
        ],
    )(x, o.reshape(B, 1), W1, b1.reshape(1, D), gamma.reshape(1, D),
      beta.reshape(1, D), W2, b2.reshape(1, D))


# speedup vs baseline: 1.0858x; 1.0858x over previous
import jax
import jax.numpy as jnp
from jax.experimental import pallas as pl
from jax.experimental.pallas import tpu as pltpu

N = 32768
B = 16
D = 128
R = 8192
T = N // R


def _body(x_hbm, o_ref, w1_ref, b1_ref, gamma_ref, beta_ref, w2_ref, b2_ref,
          out_ref, gram_ref, segsum_ref, ws_ref, f_ref, xbuf_ref, ohbuf_ref,
          sems):
    i = pl.program_id(0)
    phase_a = i < T
    t = jnp.where(phase_a, i, i - T)

    o_col = o_ref[...]
    op_col = jnp.concatenate(
        [jnp.zeros((1, 1), jnp.int32), o_col[:-1, :]], axis=0)

    @pl.when(phase_a)
    def _accum():
        @pl.when(i == 0)
        def _init():
            gram_ref[...] = jnp.zeros_like(gram_ref)
            segsum_ref[...] = jnp.zeros_like(segsum_ref)
            for k in range(T):
                pltpu.make_async_copy(
                    x_hbm.at[pl.ds(k * R, R), :],
                    xbuf_ref.at[pl.ds(k * R, R), :],
                    sems.at[k]).start()

        for k in range(T):
            @pl.when(i == k)
            def _wait():
                pltpu.make_async_copy(
                    x_hbm.at[pl.ds(k * R, R), :],
                    xbuf_ref.at[pl.ds(k * R, R), :],
                    sems.at[k]).wait()

        base = i * R
        r = jax.lax.broadcasted_iota(jnp.int32, (B, R), 1)
        oh_t = ((r >= op_col - base) & (r < o_col - base)).astype(jnp.float32)
        ohbuf_ref[:, pl.ds(i * R, R)] = oh_t
        x = xbuf_ref[pl.ds(i * R, R), :]
        gram_ref[...] += jax.lax.dot_general(
            x, x, (((0,), (0,)), ((), ())), preferred_element_type=jnp.float32)
        segsum_ref[...] += jnp.dot(oh_t, x, preferred_element_type=jnp.float32)

    @pl.when(i == T)
    def _mid():
        cnt = (o_col - op_col).astype(jnp.float32)
        segsum = segsum_ref[...]
        w1a = w1_ref[:D, :]
        seg_mean = segsum / jnp.maximum(cnt, 1.0)
        g = jax.nn.relu(jnp.dot(seg_mean, w2_ref[...],
                                preferred_element_type=jnp.float32)
                        + b2_ref[...])
        e = jnp.dot(g, w1_ref[D:, :],
                    preferred_element_type=jnp.float32) + b1_ref[...]
        segsum_a = jnp.dot(segsum, w1a, preferred_element_type=jnp.float32)
        sum_a2 = jnp.sum(w1a * jnp.dot(gram_ref[...], w1a,
                                       preferred_element_type=jnp.float32),
                         axis=0, keepdims=True)
        sum_h = jnp.sum(segsum_a + cnt * e, axis=0, keepdims=True)
        sum_h2 = sum_a2 + jnp.sum(2.0 * e * segsum_a + cnt * e * e,
                                  axis=0, keepdims=True)
        mu = sum_h / N
        var = sum_h2 / N - mu * mu
        scale = gamma_ref[...] * jax.lax.rsqrt(var + 1e-5)
        shift = beta_ref[...] - mu * scale
        ws_ref[...] = scale
        f_ref[...] = e * scale + shift

    @pl.when(jnp.logical_not(phase_a))
    def _apply():
        xb = xbuf_ref[pl.ds(t * R, R), :]
        a = jnp.dot(xb, w1_ref[:D, :], preferred_element_type=jnp.float32)
        seg_f = jax.lax.dot_general(
            ohbuf_ref[:, pl.ds(t * R, R)], f_ref[...], (((0,), (0,)), ((), ())),
            preferred_element_type=jnp.float32)
        out_ref[...] = jax.nn.relu(a * ws_ref[...] + seg_f)


def kernel(p, x, o, W1, b1, gamma, beta, W2, b2):
    del p
    full = lambda shape: pl.BlockSpec(shape, lambda *_: (0,) * len(shape))
    out_spec = pl.BlockSpec((R, D), lambda i: (jnp.where(i < T, 0, i - T), 0))

    return pl.pallas_call(
        _body,
        grid=(2 * T,),
        in_specs=[
            pl.BlockSpec(memory_space=pl.ANY),
            full((B, 1)), full((2 * D, D)), full((1, D)), full((1, D)),
            full((1, D)), full((D, D)), full((1, D)),
        ],
        out_specs=out_spec,
        out_shape=jax.ShapeDtypeStruct((N, D), jnp.float32),
        scratch_shapes=[
            pltpu.VMEM((D, D), jnp.float32),
            pltpu.VMEM((B, D), jnp.float32),
            pltpu.VMEM((1, D), jnp.float32),
            pltpu.VMEM((B, D), jnp.float32),
            pltpu.VMEM((N, D), jnp.float32),
            pltpu.VMEM((B, N), jnp.float32),
            pltpu.SemaphoreType.DMA((T,)),
        ],
    )(x, o.reshape(B, 1), W1, b1.reshape(1, D), gamma.reshape(1, D),
      beta.reshape(1, D), W2, b2.reshape(1, D))
